# baseline (speedup 1.0000x reference)
import functools

import jax
import jax.numpy as jnp
import numpy as np
from jax import lax
from jax.experimental import pallas as pl
from jax.experimental.pallas import tpu as pltpu
from jax.experimental.pallas import tpu_sc as plsc

_N = 10000
_E = 320000
_INDIM = 16
_OUTDIM = 16
_H = 256

_NC = 2
_NS = 16
_NW = _NC * _NS
_EW = _E // _NW
_GC = 2000
_NP = 10240
_RPT = _NP // _NS

_R_NP = np.kron(np.eye(_INDIM, dtype=np.float32),
                np.ones((1, _OUTDIM), dtype=np.float32))
_S_NP = np.kron(np.ones((_INDIM, 1), dtype=np.float32),
                np.eye(_OUTDIM, dtype=np.float32))
_P_NP = np.kron(np.ones((8, 8), dtype=np.float32),
                np.eye(_OUTDIM, dtype=np.float32)) / 8.0
_EK_NP = np.zeros((8, 128, 16), dtype=np.float32)
for _k in range(8):
    _EK_NP[_k, 16 * _k:16 * (_k + 1), :] = np.eye(16, dtype=np.float32)
_EKT_NP = np.ascontiguousarray(np.transpose(_EK_NP, (0, 2, 1)))
_RE_NP = np.einsum('kli,ij->klj', _EK_NP, _R_NP)
_SEK_NP = np.einsum('ij,kjl->kil', _S_NP, _EKT_NP)
_RC_NP = np.ascontiguousarray(
    np.transpose(_RE_NP, (1, 0, 2)).reshape(128, 8 * _H))
_BE = 3200

_NCH = _EW // _GC


def _gather_body(x_hbm, src2_hbm, xs_hbm,
                 idx_v, r0, r1, g0, g1, s0, s1):
    wid = lax.axis_index("s") * _NC + lax.axis_index("c")
    base = wid * _EW
    rows = (r0, r1)
    gsem = (g0, g1)
    ssem = (s0, s1)
    pltpu.sync_copy(src2_hbm.at[pl.ds(wid * _NCH, _NCH)], idx_v)
    gcp = [None] * _NCH
    scp = [None] * _NCH
    gcp[0] = pltpu.async_copy(x_hbm.at[idx_v.at[0]], rows[0], gsem[0])
    for j in range(_NCH):
        b = j % 2
        gcp[j].wait()
        if j + 1 < _NCH:
            if j >= 1:
                scp[j - 1].wait()
            gcp[j + 1] = pltpu.async_copy(
                x_hbm.at[idx_v.at[j + 1]], rows[(j + 1) % 2], gsem[(j + 1) % 2])
        scp[j] = pltpu.async_copy(
            rows[b], xs_hbm.at[pl.ds(base + j * _GC, _GC)], ssem[b])
    scp[_NCH - 1].wait()
    scp[_NCH - 2].wait()


def _scatter_body(msg_hbm, dst2_hbm, agg_hbm, idx_v, r0, r1, stage_v, acc_sh,
                  l0, l1):
    c = lax.axis_index("c")
    s = lax.axis_index("s")
    wid = s * _NC + c
    base = wid * _EW
    row0 = s * _RPT
    rows = (r0, r1)
    lsem = (l0, l1)

    def _zero_row(r, carry):
        stage_v[r] = jnp.zeros((_OUTDIM,), jnp.float32)
        return carry

    lax.fori_loop(0, _RPT, _zero_row, 0)
    pltpu.sync_copy(stage_v, acc_sh.at[pl.ds(row0, _RPT)])
    pltpu.sync_copy(dst2_hbm.at[pl.ds(wid * _NCH, _NCH)], idx_v)
    plsc.subcore_barrier()

    lcp = [None] * _NCH
    lcp[0] = pltpu.async_copy(
        msg_hbm.at[pl.ds(base, _GC)], rows[0], lsem[0])
    for j in range(_NCH):
        b = j % 2
        if j + 1 < _NCH:
            lcp[j + 1] = pltpu.async_copy(
                msg_hbm.at[pl.ds(base + (j + 1) * _GC, _GC)],
                rows[(j + 1) % 2], lsem[(j + 1) % 2])
        lcp[j].wait()
        pltpu.sync_copy(rows[b], acc_sh.at[idx_v.at[j]], add=True)

    plsc.subcore_barrier()
    pltpu.sync_copy(acc_sh.at[pl.ds(row0, _RPT)], stage_v)
    pltpu.sync_copy(stage_v, agg_hbm.at[pl.ds(c * _NP + row0, _RPT)])


@functools.lru_cache(maxsize=1)
def _sc_kernels():
    mesh = plsc.VectorSubcoreMesh(core_axis_name="c", subcore_axis_name="s")
    params = pltpu.CompilerParams(use_tc_tiling_on_sc=False)
    gather_k = pl.kernel(
        _gather_body,
        out_type=jax.ShapeDtypeStruct((_E, _INDIM), jnp.float32),
        scratch_types=[
            pltpu.VMEM((_NCH, _GC), jnp.int32),
            pltpu.VMEM((_GC, _INDIM), jnp.float32),
            pltpu.VMEM((_GC, _INDIM), jnp.float32),
            pltpu.SemaphoreType.DMA,
            pltpu.SemaphoreType.DMA,
            pltpu.SemaphoreType.DMA,
            pltpu.SemaphoreType.DMA,
        ],
        mesh=mesh,
        compiler_params=params,
    )
    scatter_k = pl.kernel(
        _scatter_body,
        out_type=jax.ShapeDtypeStruct((2 * _NP, _OUTDIM), jnp.float32),
        scratch_types=[
            pltpu.VMEM((_NCH, _GC), jnp.int32),
            pltpu.VMEM((_GC, _OUTDIM), jnp.float32),
            pltpu.VMEM((_GC, _OUTDIM), jnp.float32),
            pltpu.VMEM((_RPT, _OUTDIM), jnp.float32),
            pltpu.VMEM_SHARED((_NP, _OUTDIM), jnp.float32),
            pltpu.SemaphoreType.DMA,
            pltpu.SemaphoreType.DMA,
        ],
        mesh=mesh,
        compiler_params=params,
    )
    return gather_k, scatter_k


def _dense_body(ea, xs, w1, b1, w2, b2d, rc, sek, msg):
    eav = ea[:]
    xsv = xs[:]
    h = jnp.maximum(
        jnp.dot(eav, w1[:], preferred_element_type=jnp.float32) + b1[:],
        0.0)
    w0 = jnp.dot(h, w2[:], preferred_element_type=jnp.float32)
    xrcat = jnp.dot(xsv, rc[:], preferred_element_type=jnp.float32)
    xr = jnp.concatenate(
        [xrcat[:, 256 * k:256 * (k + 1)] for k in range(8)], axis=0)
    t = xr * w0
    q = _BE // 8
    acc = jnp.dot(xsv, b2d[:], preferred_element_type=jnp.float32)
    for k in range(8):
        acc = acc + jnp.dot(t[k * q:(k + 1) * q], sek[k],
                            preferred_element_type=jnp.float32)
    msg[:] = acc


def _final_body(agg, xp, rootb, biasv, gammav, betav, pfold, out):
    a = agg[:]
    q = _NP // 8
    n = _N // 8
    pre = (a[0:n] + a[q:q + n]
           + jnp.dot(xp[:], rootb[:], preferred_element_type=jnp.float32)
           + biasv[:])
    mu = jnp.mean(pre, axis=0, keepdims=True)
    mu_t = jnp.dot(mu, pfold[:], preferred_element_type=jnp.float32)
    d = pre - mu_t
    var = jnp.mean(d * d, axis=0, keepdims=True)
    var_t = jnp.dot(var, pfold[:], preferred_element_type=jnp.float32)
    bn = gammav[:] * d * lax.rsqrt(var_t + 1e-5) + betav[:]
    out[:] = jnp.maximum(bn, 0.0)


def kernel(x, edge_index, edge_attr, W1, b1, W2, b2, root, bias, gamma, beta):
    q = _BE // 8
    nblk = _E // _BE
    perm = lambda v: v.reshape(nblk, 8, q).transpose(0, 2, 1).reshape(-1)
    src2 = perm(edge_index[0]).reshape(_NW * _NCH, _GC)
    dst2 = perm(edge_index[1]).reshape(_NW * _NCH, _GC)
    gather_k, scatter_k = _sc_kernels()

    xs = gather_k(x, src2)
    xs_p = xs.reshape(_E // 8, 128)

    ek = jnp.asarray(_EK_NP)
    b2d = jnp.einsum('kli,ij,kjo->lo', ek,
                     b2.reshape(_INDIM, _OUTDIM),
                     jnp.asarray(_EKT_NP))

    grid = _E // _BE
    full = lambda shape: pl.BlockSpec(shape, lambda i: (0,) * len(shape))
    msg_p = pl.pallas_call(
        _dense_body,
        grid=(grid,),
        in_specs=[
            pl.BlockSpec((_BE, _INDIM), lambda i: (i, 0)),
            pl.BlockSpec((_BE // 8, 128), lambda i: (i, 0)),
            full((_INDIM, _H)),
            full((1, _H)),
            full((_H, _H)),
            full((128, 128)),
            full((128, 8 * _H)),
            full((8, _H, 128)),
        ],
        out_specs=pl.BlockSpec((_BE // 8, 128), lambda i: (i, 0)),
        out_shape=jax.ShapeDtypeStruct((_E // 8, 128), jnp.float32),
    )(edge_attr, xs_p, W1, b1.reshape(1, _H), W2, b2d,
      jnp.asarray(_RC_NP), jnp.asarray(_SEK_NP))

    msg = msg_p.reshape(_E, _OUTDIM)
    aggp = scatter_k(msg, dst2)
    agg2 = aggp.reshape(2 * _NP // 8, 128)
    xp = x.reshape(_N // 8, 128)
    rootb = jnp.kron(jnp.eye(8, dtype=jnp.float32), root)

    f0 = lambda shape: pl.BlockSpec(shape, lambda: (0, 0))
    out_p = pl.pallas_call(
        _final_body,
        in_specs=[
            f0((2 * _NP // 8, 128)),
            f0((_N // 8, 128)),
            f0((128, 128)),
            f0((1, 128)),
            f0((1, 128)),
            f0((1, 128)),
            f0((128, 128)),
        ],
        out_specs=f0((_N // 8, 128)),
        out_shape=jax.ShapeDtypeStruct((_N // 8, 128), jnp.float32),
    )(agg2, xp, rootb, jnp.tile(bias, 8).reshape(1, 128),
      jnp.tile(gamma, 8).reshape(1, 128), jnp.tile(beta, 8).reshape(1, 128),
      jnp.asarray(_P_NP))
    return out_p.reshape(_N, _OUTDIM)


# speedup vs baseline: 1.2021x; 1.2021x over previous
import functools

import jax
import jax.numpy as jnp
import numpy as np
from jax import lax
from jax.experimental import pallas as pl
from jax.experimental.pallas import tpu as pltpu
from jax.experimental.pallas import tpu_sc as plsc

_N = 10000
_E = 320000
_INDIM = 16
_OUTDIM = 16
_H = 256

_NC = 2
_NS = 16
_NW = _NC * _NS
_EW = _E // _NW
_GC = 2000
_NP = 10240
_RPT = _NP // _NS

_R_NP = np.kron(np.eye(_INDIM, dtype=np.float32),
                np.ones((1, _OUTDIM), dtype=np.float32))
_S_NP = np.kron(np.ones((_INDIM, 1), dtype=np.float32),
                np.eye(_OUTDIM, dtype=np.float32))
_P_NP = np.kron(np.ones((8, 8), dtype=np.float32),
                np.eye(_OUTDIM, dtype=np.float32)) / 8.0
_EK_NP = np.zeros((8, 128, 16), dtype=np.float32)
for _k in range(8):
    _EK_NP[_k, 16 * _k:16 * (_k + 1), :] = np.eye(16, dtype=np.float32)
_EKT_NP = np.ascontiguousarray(np.transpose(_EK_NP, (0, 2, 1)))
_RE_NP = np.einsum('kli,ij->klj', _EK_NP, _R_NP)
_SEK_NP = np.einsum('ij,kjl->kil', _S_NP, _EKT_NP)
_RC_NP = np.ascontiguousarray(
    np.transpose(_RE_NP, (1, 0, 2)).reshape(128, 8 * _H))
_BE = 3200

_NCH = _EW // _GC


def _gather_body(x_hbm, src2_hbm, xs_hbm,
                 idx_v, r0, r1, g0, g1, s0, s1):
    wid = lax.axis_index("s") * _NC + lax.axis_index("c")
    base = wid * _EW
    rows = (r0, r1)
    gsem = (g0, g1)
    ssem = (s0, s1)
    pltpu.sync_copy(src2_hbm.at[pl.ds(wid * _NCH, _NCH)], idx_v)
    gcp = [None] * _NCH
    scp = [None] * _NCH
    gcp[0] = pltpu.async_copy(x_hbm.at[idx_v.at[0]], rows[0], gsem[0])
    for j in range(_NCH):
        b = j % 2
        gcp[j].wait()
        if j + 1 < _NCH:
            if j >= 1:
                scp[j - 1].wait()
            gcp[j + 1] = pltpu.async_copy(
                x_hbm.at[idx_v.at[j + 1]], rows[(j + 1) % 2], gsem[(j + 1) % 2])
        scp[j] = pltpu.async_copy(
            rows[b], xs_hbm.at[pl.ds(base + j * _GC, _GC)], ssem[b])
    scp[_NCH - 1].wait()
    scp[_NCH - 2].wait()


def _scatter_body(msg_hbm, dst2_hbm, agg_hbm, idx_v, r0, r1, stage_v, acc_sh,
                  l0, l1):
    c = lax.axis_index("c")
    s = lax.axis_index("s")
    wid = s * _NC + c
    base = wid * _EW
    row0 = s * _RPT
    rows = (r0, r1)
    lsem = (l0, l1)

    def _zero_row(r, carry):
        stage_v[r] = jnp.zeros((_OUTDIM,), jnp.float32)
        return carry

    lax.fori_loop(0, _RPT, _zero_row, 0)
    pltpu.sync_copy(stage_v, acc_sh.at[pl.ds(row0, _RPT)])
    pltpu.sync_copy(dst2_hbm.at[pl.ds(wid * _NCH, _NCH)], idx_v)
    plsc.subcore_barrier()

    lcp = [None] * _NCH
    lcp[0] = pltpu.async_copy(
        msg_hbm.at[pl.ds(base, _GC)], rows[0], lsem[0])
    for j in range(_NCH):
        b = j % 2
        if j + 1 < _NCH:
            lcp[j + 1] = pltpu.async_copy(
                msg_hbm.at[pl.ds(base + (j + 1) * _GC, _GC)],
                rows[(j + 1) % 2], lsem[(j + 1) % 2])
        lcp[j].wait()
        pltpu.sync_copy(rows[b], acc_sh.at[idx_v.at[j]], add=True)

    plsc.subcore_barrier()
    pltpu.sync_copy(acc_sh.at[pl.ds(row0, _RPT)], stage_v)
    pltpu.sync_copy(stage_v, agg_hbm.at[pl.ds(c * _NP + row0, _RPT)])


@functools.lru_cache(maxsize=1)
def _sc_kernels():
    mesh = plsc.VectorSubcoreMesh(core_axis_name="c", subcore_axis_name="s")
    params = pltpu.CompilerParams(use_tc_tiling_on_sc=False)
    gather_k = pl.kernel(
        _gather_body,
        out_type=jax.ShapeDtypeStruct((_E, _INDIM), jnp.float32),
        scratch_types=[
            pltpu.VMEM((_NCH, _GC), jnp.int32),
            pltpu.VMEM((_GC, _INDIM), jnp.float32),
            pltpu.VMEM((_GC, _INDIM), jnp.float32),
            pltpu.SemaphoreType.DMA,
            pltpu.SemaphoreType.DMA,
            pltpu.SemaphoreType.DMA,
            pltpu.SemaphoreType.DMA,
        ],
        mesh=mesh,
        compiler_params=params,
    )
    scatter_k = pl.kernel(
        _scatter_body,
        out_type=jax.ShapeDtypeStruct((2 * _NP, _OUTDIM), jnp.float32),
        scratch_types=[
            pltpu.VMEM((_NCH, _GC), jnp.int32),
            pltpu.VMEM((_GC, _OUTDIM), jnp.float32),
            pltpu.VMEM((_GC, _OUTDIM), jnp.float32),
            pltpu.VMEM((_RPT, _OUTDIM), jnp.float32),
            pltpu.VMEM_SHARED((_NP, _OUTDIM), jnp.float32),
            pltpu.SemaphoreType.DMA,
            pltpu.SemaphoreType.DMA,
        ],
        mesh=mesh,
        compiler_params=params,
    )
    return gather_k, scatter_k


def _dense_body(ea, xs, w1, b1, w2, b2d, rc, sek, msg):
    eav = ea[:]
    xsv = xs[:]
    hcat = jnp.dot(eav, w1[:], preferred_element_type=jnp.float32)
    h = jnp.concatenate(
        [hcat[:, 256 * k:256 * (k + 1)] for k in range(8)], axis=0)
    h = jnp.maximum(h + b1[:], 0.0)
    w0 = jnp.dot(h, w2[:], preferred_element_type=jnp.float32)
    xrcat = jnp.dot(xsv, rc[:], preferred_element_type=jnp.float32)
    xr = jnp.concatenate(
        [xrcat[:, 256 * k:256 * (k + 1)] for k in range(8)], axis=0)
    t = xr * w0
    q = _BE // 8
    acc = jnp.dot(xsv, b2d[:], preferred_element_type=jnp.float32)
    for k in range(8):
        acc = acc + jnp.dot(t[k * q:(k + 1) * q], sek[k],
                            preferred_element_type=jnp.float32)
    msg[:] = acc


def _final_body(agg, xp, rootb, biasv, gammav, betav, pfold, out):
    a = agg[:]
    q = _NP // 8
    n = _N // 8
    pre = (a[0:n] + a[q:q + n]
           + jnp.dot(xp[:], rootb[:], preferred_element_type=jnp.float32)
           + biasv[:])
    mu = jnp.mean(pre, axis=0, keepdims=True)
    mu_t = jnp.dot(mu, pfold[:], preferred_element_type=jnp.float32)
    d = pre - mu_t
    var = jnp.mean(d * d, axis=0, keepdims=True)
    var_t = jnp.dot(var, pfold[:], preferred_element_type=jnp.float32)
    bn = gammav[:] * d * lax.rsqrt(var_t + 1e-5) + betav[:]
    out[:] = jnp.maximum(bn, 0.0)


def kernel(x, edge_index, edge_attr, W1, b1, W2, b2, root, bias, gamma, beta):
    src2 = edge_index[0].reshape(_NW * _NCH, _GC)
    dst2 = edge_index[1].reshape(_NW * _NCH, _GC)
    gather_k, scatter_k = _sc_kernels()

    xs = gather_k(x, src2)
    xs_p = xs.reshape(_E // 8, 128)
    ea_p = edge_attr.reshape(_E // 8, 128)

    ek = jnp.asarray(_EK_NP)
    w1e = jnp.einsum('kli,ij->klj', ek, W1)
    w1c = jnp.transpose(w1e, (1, 0, 2)).reshape(128, 8 * _H)
    b2d = jnp.einsum('kli,ij,kjo->lo', ek,
                     b2.reshape(_INDIM, _OUTDIM),
                     jnp.asarray(_EKT_NP))

    grid = _E // _BE
    full = lambda shape: pl.BlockSpec(shape, lambda i: (0,) * len(shape))
    msg_p = pl.pallas_call(
        _dense_body,
        grid=(grid,),
        in_specs=[
            pl.BlockSpec((_BE // 8, 128), lambda i: (i, 0)),
            pl.BlockSpec((_BE // 8, 128), lambda i: (i, 0)),
            full((128, 8 * _H)),
            full((1, _H)),
            full((_H, _H)),
            full((128, 128)),
            full((128, 8 * _H)),
            full((8, _H, 128)),
        ],
        out_specs=pl.BlockSpec((_BE // 8, 128), lambda i: (i, 0)),
        out_shape=jax.ShapeDtypeStruct((_E // 8, 128), jnp.float32),
    )(ea_p, xs_p, w1c, b1.reshape(1, _H), W2, b2d,
      jnp.asarray(_RC_NP), jnp.asarray(_SEK_NP))

    msg = msg_p.reshape(_E, _OUTDIM)
    aggp = scatter_k(msg, dst2)
    agg2 = aggp.reshape(2 * _NP // 8, 128)
    xp = x.reshape(_N // 8, 128)
    rootb = jnp.kron(jnp.eye(8, dtype=jnp.float32), root)

    f0 = lambda shape: pl.BlockSpec(shape, lambda: (0, 0))
    out_p = pl.pallas_call(
        _final_body,
        in_specs=[
            f0((2 * _NP // 8, 128)),
            f0((_N // 8, 128)),
            f0((128, 128)),
            f0((1, 128)),
            f0((1, 128)),
            f0((1, 128)),
            f0((128, 128)),
        ],
        out_specs=f0((_N // 8, 128)),
        out_shape=jax.ShapeDtypeStruct((_N // 8, 128), jnp.float32),
    )(agg2, xp, rootb, jnp.tile(bias, 8).reshape(1, 128),
      jnp.tile(gamma, 8).reshape(1, 128), jnp.tile(beta, 8).reshape(1, 128),
      jnp.asarray(_P_NP))
    return out_p.reshape(_N, _OUTDIM)
